# baseline (speedup 1.0000x reference)
import jax
import jax.numpy as jnp
from jax import lax
from jax.experimental import pallas as pl
from jax.experimental.pallas import tpu as pltpu
from jax.experimental.pallas import tpu_sc as plsc

DIM = 768
E = 8
H = 1536
K = 2
T = 2048

NC = 2
NS = 16
NW = NC * NS
L = 16
TPW = T // NW

BR = 256
BRLOG = 8
NB = 24
NR = NB * BR


def _gating_kernel(flat_ref, gate_w_ref, i1_ref, i2_ref, c1_ref, c2_ref,
                   loss_ref, wcounts_ref, bexp_ref, bact_ref, bsrc_ref):
    flat = flat_ref[...]
    logits = jnp.dot(flat, gate_w_ref[...], preferred_element_type=jnp.float32)
    m = jnp.max(logits, axis=-1, keepdims=True)
    ex = jnp.exp(logits - m)
    gates = ex / jnp.sum(ex, axis=-1, keepdims=True)

    eidx = jax.lax.broadcasted_iota(jnp.int32, gates.shape, 1)
    i1 = jnp.argmax(gates, axis=-1, keepdims=True)
    g1 = jnp.max(gates, axis=-1, keepdims=True)
    masked = jnp.where(eidx == i1, -jnp.inf, gates)
    i2 = jnp.argmax(masked, axis=-1, keepdims=True)
    g2 = jnp.max(masked, axis=-1, keepdims=True)

    denom = g1 + g2
    i1_ref[...] = i1.astype(jnp.int32)
    i2_ref[...] = i2.astype(jnp.int32)
    c1_ref[...] = g1 / denom
    c2_ref[...] = g2 / denom

    sel = ((eidx == i1) | (eidx == i2)).astype(jnp.float32)
    usage = jnp.sum(sel, axis=0) / gates.shape[0]
    loss = jnp.mean((usage - 1.0 / E) ** 2) * E * 0.01
    loss_ref[...] = jnp.reshape(loss, (1, 1))

    eidx16 = jax.lax.broadcasted_iota(jnp.int32, (T, L), 1)
    sel16 = ((eidx16 == i1) | (eidx16 == i2)).astype(jnp.float32)
    wc = jnp.sum(sel16.reshape(NW, TPW, L), axis=1)
    wcounts_ref[...] = wc.astype(jnp.int32)

    total = jnp.sum(wc, axis=0, keepdims=True)
    nblk = jnp.floor((total + (BR - 1)) * (1.0 / BR))
    rr = jax.lax.broadcasted_iota(jnp.int32, (L, L), 0)
    cc = jax.lax.broadcasted_iota(jnp.int32, (L, L), 1)
    tril = (rr < cc).astype(jnp.float32)
    cume = jnp.dot(nblk, tril, preferred_element_type=jnp.float32)
    used = jnp.sum(nblk, axis=-1, keepdims=True)
    ii = jax.lax.broadcasted_iota(jnp.int32, (NB, 1), 0).astype(jnp.float32)
    ge = (ii >= cume).astype(jnp.float32)
    bexp_ref[...] = (jnp.sum(ge[:, :E], axis=1, keepdims=True) - 1.0
                     ).astype(jnp.int32)
    bact_ref[...] = (ii < used).astype(jnp.int32)
    bsrc_ref[...] = jnp.minimum(ii, used - 1.0).astype(jnp.int32)


def _sc_wid():
    return lax.axis_index("s") * NC + lax.axis_index("c")


def _lanes():
    return lax.broadcasted_iota(jnp.int32, (L,), 0)


def _extract(vec, lane):
    return jnp.sum(jnp.where(_lanes() == lane, vec, vec.dtype.type(0)))


def _b1b_kernel(i1_hbm, i2_hbm, x_hbm, wcounts_hbm,
                xg_hbm, pos1_hbm, pos2_hbm,
                i1_v, i2_v, cnt_all_v, pos1_v, pos2_v,
                xrows_v, sem):
    wid = _sc_wid()
    base = wid * TPW
    lanes = _lanes()

    cps = [
        pltpu.async_copy(wcounts_hbm, cnt_all_v, sem),
        pltpu.async_copy(i1_hbm.at[pl.ds(base, TPW)], i1_v, sem),
        pltpu.async_copy(i2_hbm.at[pl.ds(base, TPW)], i2_v, sem),
        pltpu.async_copy(x_hbm.at[pl.ds(base, TPW)], xrows_v, sem),
    ]
    for cp in cps:
        cp.wait()

    total = jnp.zeros((L,), jnp.int32)
    pref = jnp.zeros((L,), jnp.int32)
    for w in range(NW):
        row = cnt_all_v[w]
        total = total + row
        pref = pref + jnp.where(wid > w, row, 0)
    nblk = (total + (BR - 1)) >> BRLOG
    cum = plsc.cumsum(nblk)
    cume = cum - nblk
    off = cume * BR + pref

    def assign(src_ref, pos_ref, off):
        for v in range(TPW // L):
            vec = src_ref[pl.ds(v * L, L)]
            pos = jnp.zeros((L,), jnp.int32)
            for e in range(E):
                m = vec == e
                mi = jnp.where(m, 1, 0)
                pr = plsc.cumsum(mi)
                base_e = _extract(off, e)
                pos = jnp.where(m, base_e + pr - 1, pos)
                off = off + jnp.where(lanes == e, jnp.sum(mi), 0)
            pos_ref[pl.ds(v * L, L)] = pos
        return off

    off = assign(i1_v, pos1_v, off)
    off = assign(i2_v, pos2_v, off)

    cps = [
        pltpu.async_copy(pos1_v, pos1_hbm.at[pl.ds(base, TPW)], sem),
        pltpu.async_copy(pos2_v, pos2_hbm.at[pl.ds(base, TPW)], sem),
        pltpu.async_copy(xrows_v, xg_hbm.at[pos1_v], sem),
        pltpu.async_copy(xrows_v, xg_hbm.at[pos2_v], sem),
    ]
    for cp in cps:
        cp.wait()


def _ffn_kernel(be_ref, act_ref, bsrc_ref, xg_ref, w1_ref, b1_ref, w2_ref,
                b2_ref, out_ref, w1b_ref, w2b_ref):
    i = pl.program_id(0)
    prev = be_ref[jnp.maximum(i - 1, 0)]

    @pl.when(jnp.logical_or(i == 0, be_ref[i] != prev))
    def _():
        w1b_ref[...] = w1_ref[0].astype(jnp.bfloat16)
        w2b_ref[...] = w2_ref[0].astype(jnp.bfloat16)

    @pl.when(act_ref[i] == 1)
    def _():
        x = xg_ref[...].astype(jnp.bfloat16)
        h = jnp.dot(x, w1b_ref[...], preferred_element_type=jnp.float32)
        h = h + b1_ref[0]
        h = 0.5 * h * (1.0 + lax.erf(h * 0.7071067811865476))
        eo = jnp.dot(h.astype(jnp.bfloat16), w2b_ref[...],
                     preferred_element_type=jnp.float32)
        out_ref[...] = eo + b2_ref[0]


def _comb_kernel(pos1_hbm, pos2_hbm, c1_hbm, c2_hbm, outg_hbm, out_hbm,
                 pos1_v, pos2_v, c1_v, c2_v, rows1_v, rows2_v, sem):
    wid = _sc_wid()
    base = wid * TPW
    lanes = _lanes()
    cps = [
        pltpu.async_copy(pos1_hbm.at[pl.ds(base, TPW)], pos1_v, sem),
        pltpu.async_copy(pos2_hbm.at[pl.ds(base, TPW)], pos2_v, sem),
        pltpu.async_copy(c1_hbm.at[pl.ds(base, TPW)], c1_v, sem),
        pltpu.async_copy(c2_hbm.at[pl.ds(base, TPW)], c2_v, sem),
    ]
    for cp in cps:
        cp.wait()
    HF = TPW // 2
    gathers = []
    for h in range(2):
        hs = pl.ds(h * HF, HF)
        gathers.append((
            pltpu.async_copy(outg_hbm.at[pos1_v.at[hs]], rows1_v.at[hs], sem),
            pltpu.async_copy(outg_hbm.at[pos2_v.at[hs]], rows2_v.at[hs], sem),
        ))

    def body(j, _):
        grp = pl.ds((j // L) * L, L)
        lane = j % L
        s1 = jnp.sum(jnp.where(lanes == lane, c1_v[grp], 0.0))
        s2 = jnp.sum(jnp.where(lanes == lane, c2_v[grp], 0.0))
        for v in range(DIM // L):
            sl = pl.ds(v * L, L)
            rows1_v[j, sl] = rows1_v[j, sl] * s1 + rows2_v[j, sl] * s2
        return 0

    outs = []
    for h in range(2):
        g1, g2 = gathers[h]
        g1.wait()
        g2.wait()
        lax.fori_loop(h * HF, (h + 1) * HF, body, 0)
        outs.append(pltpu.async_copy(
            rows1_v.at[pl.ds(h * HF, HF)],
            out_hbm.at[pl.ds(base + h * HF, HF)], sem))
    for cp in outs:
        cp.wait()


@jax.jit
def kernel(x, gate_w, w1, b1, w2, b2):
    Bt, S, D = x.shape
    flat = x.reshape(T, D)

    i1, i2, c1, c2, loss, wcounts, bexp, bact, bsrc = pl.pallas_call(
        _gating_kernel,
        out_shape=(
            jax.ShapeDtypeStruct((T, 1), jnp.int32),
            jax.ShapeDtypeStruct((T, 1), jnp.int32),
            jax.ShapeDtypeStruct((T, 1), jnp.float32),
            jax.ShapeDtypeStruct((T, 1), jnp.float32),
            jax.ShapeDtypeStruct((1, 1), jnp.float32),
            jax.ShapeDtypeStruct((NW, L), jnp.int32),
            jax.ShapeDtypeStruct((NB, 1), jnp.int32),
            jax.ShapeDtypeStruct((NB, 1), jnp.int32),
            jax.ShapeDtypeStruct((NB, 1), jnp.int32),
        ),
    )(flat, gate_w)
    i1 = i1.reshape(T)
    i2 = i2.reshape(T)
    c1 = c1.reshape(T)
    c2 = c2.reshape(T)
    bexp = bexp.reshape(NB)
    bact = bact.reshape(NB)
    bsrc = bsrc.reshape(NB)

    mesh = plsc.VectorSubcoreMesh(core_axis_name="c", subcore_axis_name="s")
    sc_params = pltpu.CompilerParams(needs_layout_passes=False)

    xg, pos1, pos2 = pl.kernel(
        _b1b_kernel,
        out_type=(
            jax.ShapeDtypeStruct((NR, DIM), jnp.float32),
            jax.ShapeDtypeStruct((T,), jnp.int32),
            jax.ShapeDtypeStruct((T,), jnp.int32),
        ),
        mesh=mesh,
        compiler_params=sc_params,
        scratch_types=[
            pltpu.VMEM((TPW,), jnp.int32),
            pltpu.VMEM((TPW,), jnp.int32),
            pltpu.VMEM((NW, L), jnp.int32),
            pltpu.VMEM((TPW,), jnp.int32),
            pltpu.VMEM((TPW,), jnp.int32),
            pltpu.VMEM((TPW, DIM), jnp.float32),
            pltpu.SemaphoreType.DMA,
        ],
    )(i1, i2, flat, wcounts)

    grid_spec = pltpu.PrefetchScalarGridSpec(
        num_scalar_prefetch=3,
        grid=(NB,),
        in_specs=[
            pl.BlockSpec((BR, DIM), lambda i, be, act, bs: (bs[i], 0)),
            pl.BlockSpec((1, DIM, H), lambda i, be, act, bs: (be[i], 0, 0)),
            pl.BlockSpec((1, 1, H), lambda i, be, act, bs: (be[i], 0, 0)),
            pl.BlockSpec((1, H, DIM), lambda i, be, act, bs: (be[i], 0, 0)),
            pl.BlockSpec((1, 1, DIM), lambda i, be, act, bs: (be[i], 0, 0)),
        ],
        out_specs=pl.BlockSpec((BR, DIM), lambda i, be, act, bs: (bs[i], 0)),
        scratch_shapes=[
            pltpu.VMEM((DIM, H), jnp.bfloat16),
            pltpu.VMEM((H, DIM), jnp.bfloat16),
        ],
    )
    outg = pl.pallas_call(
        _ffn_kernel,
        grid_spec=grid_spec,
        out_shape=jax.ShapeDtypeStruct((NR, DIM), jnp.float32),
    )(bexp, bact, bsrc, xg, w1, b1.reshape(E, 1, H), w2, b2.reshape(E, 1, DIM))

    out2d = pl.kernel(
        _comb_kernel,
        out_type=jax.ShapeDtypeStruct((T, DIM), jnp.float32),
        mesh=mesh,
        compiler_params=sc_params,
        scratch_types=[
            pltpu.VMEM((TPW,), jnp.int32),
            pltpu.VMEM((TPW,), jnp.int32),
            pltpu.VMEM((TPW,), jnp.float32),
            pltpu.VMEM((TPW,), jnp.float32),
            pltpu.VMEM((TPW, DIM), jnp.float32),
            pltpu.VMEM((TPW, DIM), jnp.float32),
            pltpu.SemaphoreType.DMA,
        ],
    )(pos1, pos2, c1, c2, outg)

    return out2d.reshape(Bt, S, D), loss[0, 0]


# speedup vs baseline: 1.0368x; 1.0368x over previous
import jax
import jax.numpy as jnp
from jax import lax
from jax.experimental import pallas as pl
from jax.experimental.pallas import tpu as pltpu
from jax.experimental.pallas import tpu_sc as plsc

DIM = 768
E = 8
H = 1536
K = 2
T = 2048

NC = 2
NS = 16
NW = NC * NS
L = 16
TPW = T // NW

BR = 256
BRLOG = 8
NB = 24
NR = NB * BR


def _gating_kernel(flat_ref, gate_w_ref, i1_ref, i2_ref, c1_ref, c2_ref,
                   loss_ref, wcounts_ref, bexp_ref, bact_ref, bsrc_ref):
    flat = flat_ref[...]
    logits = jnp.dot(flat, gate_w_ref[...], preferred_element_type=jnp.float32)
    m = jnp.max(logits, axis=-1, keepdims=True)
    ex = jnp.exp(logits - m)
    gates = ex / jnp.sum(ex, axis=-1, keepdims=True)

    eidx = jax.lax.broadcasted_iota(jnp.int32, gates.shape, 1)
    i1 = jnp.argmax(gates, axis=-1, keepdims=True)
    g1 = jnp.max(gates, axis=-1, keepdims=True)
    masked = jnp.where(eidx == i1, -jnp.inf, gates)
    i2 = jnp.argmax(masked, axis=-1, keepdims=True)
    g2 = jnp.max(masked, axis=-1, keepdims=True)

    denom = g1 + g2
    i1_ref[...] = i1.astype(jnp.int32)
    i2_ref[...] = i2.astype(jnp.int32)
    c1_ref[...] = g1 / denom
    c2_ref[...] = g2 / denom

    sel = ((eidx == i1) | (eidx == i2)).astype(jnp.float32)
    usage = jnp.sum(sel, axis=0) / gates.shape[0]
    loss = jnp.mean((usage - 1.0 / E) ** 2) * E * 0.01
    loss_ref[...] = jnp.reshape(loss, (1, 1))

    eidx16 = jax.lax.broadcasted_iota(jnp.int32, (T, L), 1)
    sel16 = ((eidx16 == i1) | (eidx16 == i2)).astype(jnp.float32)
    wc = jnp.sum(sel16.reshape(NW, TPW, L), axis=1)
    wcounts_ref[...] = wc.astype(jnp.int32)

    total = jnp.sum(wc, axis=0, keepdims=True)
    nblk = jnp.floor((total + (BR - 1)) * (1.0 / BR))
    rr = jax.lax.broadcasted_iota(jnp.int32, (L, L), 0)
    cc = jax.lax.broadcasted_iota(jnp.int32, (L, L), 1)
    tril = (rr < cc).astype(jnp.float32)
    cume = jnp.dot(nblk, tril, preferred_element_type=jnp.float32)
    used = jnp.sum(nblk, axis=-1, keepdims=True)
    ii = jax.lax.broadcasted_iota(jnp.int32, (NB, 1), 0).astype(jnp.float32)
    ge = (ii >= cume).astype(jnp.float32)
    bexp_ref[...] = (jnp.sum(ge[:, :E], axis=1, keepdims=True) - 1.0
                     ).astype(jnp.int32)
    bact_ref[...] = (ii < used).astype(jnp.int32)
    bsrc_ref[...] = jnp.minimum(ii, used - 1.0).astype(jnp.int32)


def _sc_wid():
    return lax.axis_index("s") * NC + lax.axis_index("c")


def _lanes():
    return lax.broadcasted_iota(jnp.int32, (L,), 0)


def _extract(vec, lane):
    return jnp.sum(jnp.where(_lanes() == lane, vec, vec.dtype.type(0)))


def _b1b_kernel(i1_hbm, i2_hbm, x_hbm, wcounts_hbm,
                xg_hbm, pos1_hbm, pos2_hbm,
                i1_v, i2_v, cnt_all_v, pos1_v, pos2_v,
                xrows_v, sem):
    wid = _sc_wid()
    base = wid * TPW
    lanes = _lanes()

    cps = [
        pltpu.async_copy(wcounts_hbm, cnt_all_v, sem),
        pltpu.async_copy(i1_hbm.at[pl.ds(base, TPW)], i1_v, sem),
        pltpu.async_copy(i2_hbm.at[pl.ds(base, TPW)], i2_v, sem),
        pltpu.async_copy(x_hbm.at[pl.ds(base, TPW)], xrows_v, sem),
    ]
    for cp in cps:
        cp.wait()

    total = jnp.zeros((L,), jnp.int32)
    pref = jnp.zeros((L,), jnp.int32)
    for w in range(NW):
        row = cnt_all_v[w]
        total = total + row
        pref = pref + jnp.where(wid > w, row, 0)
    nblk = (total + (BR - 1)) >> BRLOG
    cum = plsc.cumsum(nblk)
    cume = cum - nblk
    off = cume * BR + pref

    def assign(src_ref, pos_ref, off):
        for v in range(TPW // L):
            vec = src_ref[pl.ds(v * L, L)]
            pos = jnp.zeros((L,), jnp.int32)
            for e in range(E):
                m = vec == e
                mi = jnp.where(m, 1, 0)
                pr = plsc.cumsum(mi)
                base_e = _extract(off, e)
                pos = jnp.where(m, base_e + pr - 1, pos)
                off = off + jnp.where(lanes == e, jnp.sum(mi), 0)
            pos_ref[pl.ds(v * L, L)] = pos
        return off

    off = assign(i1_v, pos1_v, off)
    off = assign(i2_v, pos2_v, off)

    cps = [
        pltpu.async_copy(pos1_v, pos1_hbm.at[pl.ds(base, TPW)], sem),
        pltpu.async_copy(pos2_v, pos2_hbm.at[pl.ds(base, TPW)], sem),
        pltpu.async_copy(xrows_v, xg_hbm.at[pos1_v], sem),
        pltpu.async_copy(xrows_v, xg_hbm.at[pos2_v], sem),
    ]
    for cp in cps:
        cp.wait()


def _ffn_kernel(be_ref, act_ref, bsrc_ref, xg_ref, w1_ref, b1_ref, w2_ref,
                b2_ref, out_ref):
    i = pl.program_id(0)

    @pl.when(act_ref[i] == 1)
    def _():
        x = xg_ref[...]
        h = jnp.dot(x, w1_ref[0], preferred_element_type=jnp.float32)
        h = h + b1_ref[0]
        h = 0.5 * h * (1.0 + lax.erf(h * 0.7071067811865476))
        eo = jnp.dot(h, w2_ref[0], preferred_element_type=jnp.float32)
        out_ref[...] = eo + b2_ref[0]


def _comb_kernel(pos1_hbm, pos2_hbm, c1_hbm, c2_hbm, outg_hbm, out_hbm,
                 pos1_v, pos2_v, c1_v, c2_v, rows1_v, rows2_v, sem):
    wid = _sc_wid()
    base = wid * TPW
    lanes = _lanes()
    cps = [
        pltpu.async_copy(pos1_hbm.at[pl.ds(base, TPW)], pos1_v, sem),
        pltpu.async_copy(pos2_hbm.at[pl.ds(base, TPW)], pos2_v, sem),
        pltpu.async_copy(c1_hbm.at[pl.ds(base, TPW)], c1_v, sem),
        pltpu.async_copy(c2_hbm.at[pl.ds(base, TPW)], c2_v, sem),
    ]
    for cp in cps:
        cp.wait()
    HF = TPW // 2
    gathers = []
    for h in range(2):
        hs = pl.ds(h * HF, HF)
        gathers.append((
            pltpu.async_copy(outg_hbm.at[pos1_v.at[hs]], rows1_v.at[hs], sem),
            pltpu.async_copy(outg_hbm.at[pos2_v.at[hs]], rows2_v.at[hs], sem),
        ))

    def body(j, _):
        grp = pl.ds((j // L) * L, L)
        lane = j % L
        s1 = jnp.sum(jnp.where(lanes == lane, c1_v[grp], 0.0))
        s2 = jnp.sum(jnp.where(lanes == lane, c2_v[grp], 0.0))
        for v in range(DIM // L):
            sl = pl.ds(v * L, L)
            rows1_v[j, sl] = rows1_v[j, sl] * s1 + rows2_v[j, sl] * s2
        return 0

    outs = []
    for h in range(2):
        g1, g2 = gathers[h]
        g1.wait()
        g2.wait()
        lax.fori_loop(h * HF, (h + 1) * HF, body, 0)
        outs.append(pltpu.async_copy(
            rows1_v.at[pl.ds(h * HF, HF)],
            out_hbm.at[pl.ds(base + h * HF, HF)], sem))
    for cp in outs:
        cp.wait()


@jax.jit
def kernel(x, gate_w, w1, b1, w2, b2):
    Bt, S, D = x.shape
    flat = x.reshape(T, D)

    i1, i2, c1, c2, loss, wcounts, bexp, bact, bsrc = pl.pallas_call(
        _gating_kernel,
        out_shape=(
            jax.ShapeDtypeStruct((T, 1), jnp.int32),
            jax.ShapeDtypeStruct((T, 1), jnp.int32),
            jax.ShapeDtypeStruct((T, 1), jnp.float32),
            jax.ShapeDtypeStruct((T, 1), jnp.float32),
            jax.ShapeDtypeStruct((1, 1), jnp.float32),
            jax.ShapeDtypeStruct((NW, L), jnp.int32),
            jax.ShapeDtypeStruct((NB, 1), jnp.int32),
            jax.ShapeDtypeStruct((NB, 1), jnp.int32),
            jax.ShapeDtypeStruct((NB, 1), jnp.int32),
        ),
    )(flat, gate_w)
    i1 = i1.reshape(T)
    i2 = i2.reshape(T)
    c1 = c1.reshape(T)
    c2 = c2.reshape(T)
    bexp = bexp.reshape(NB)
    bact = bact.reshape(NB)
    bsrc = bsrc.reshape(NB)

    mesh = plsc.VectorSubcoreMesh(core_axis_name="c", subcore_axis_name="s")
    sc_params = pltpu.CompilerParams(needs_layout_passes=False)

    xg, pos1, pos2 = pl.kernel(
        _b1b_kernel,
        out_type=(
            jax.ShapeDtypeStruct((NR, DIM), jnp.float32),
            jax.ShapeDtypeStruct((T,), jnp.int32),
            jax.ShapeDtypeStruct((T,), jnp.int32),
        ),
        mesh=mesh,
        compiler_params=sc_params,
        scratch_types=[
            pltpu.VMEM((TPW,), jnp.int32),
            pltpu.VMEM((TPW,), jnp.int32),
            pltpu.VMEM((NW, L), jnp.int32),
            pltpu.VMEM((TPW,), jnp.int32),
            pltpu.VMEM((TPW,), jnp.int32),
            pltpu.VMEM((TPW, DIM), jnp.float32),
            pltpu.SemaphoreType.DMA,
        ],
    )(i1, i2, flat, wcounts)

    grid_spec = pltpu.PrefetchScalarGridSpec(
        num_scalar_prefetch=3,
        grid=(NB,),
        in_specs=[
            pl.BlockSpec((BR, DIM), lambda i, be, act, bs: (bs[i], 0)),
            pl.BlockSpec((1, DIM, H), lambda i, be, act, bs: (be[i], 0, 0)),
            pl.BlockSpec((1, 1, H), lambda i, be, act, bs: (be[i], 0, 0)),
            pl.BlockSpec((1, H, DIM), lambda i, be, act, bs: (be[i], 0, 0)),
            pl.BlockSpec((1, 1, DIM), lambda i, be, act, bs: (be[i], 0, 0)),
        ],
        out_specs=pl.BlockSpec((BR, DIM), lambda i, be, act, bs: (bs[i], 0)),
    )
    outg = pl.pallas_call(
        _ffn_kernel,
        grid_spec=grid_spec,
        out_shape=jax.ShapeDtypeStruct((NR, DIM), jnp.float32),
    )(bexp, bact, bsrc, xg, w1, b1.reshape(E, 1, H), w2, b2.reshape(E, 1, DIM))

    out2d = pl.kernel(
        _comb_kernel,
        out_type=jax.ShapeDtypeStruct((T, DIM), jnp.float32),
        mesh=mesh,
        compiler_params=sc_params,
        scratch_types=[
            pltpu.VMEM((TPW,), jnp.int32),
            pltpu.VMEM((TPW,), jnp.int32),
            pltpu.VMEM((TPW,), jnp.float32),
            pltpu.VMEM((TPW,), jnp.float32),
            pltpu.VMEM((TPW, DIM), jnp.float32),
            pltpu.VMEM((TPW, DIM), jnp.float32),
            pltpu.SemaphoreType.DMA,
        ],
    )(pos1, pos2, c1, c2, outg)

    return out2d.reshape(Bt, S, D), loss[0, 0]


# speedup vs baseline: 1.0640x; 1.0262x over previous
import jax
import jax.numpy as jnp
from jax import lax
from jax.experimental import pallas as pl
from jax.experimental.pallas import tpu as pltpu
from jax.experimental.pallas import tpu_sc as plsc

DIM = 768
E = 8
H = 1536
K = 2
T = 2048

NC = 2
NS = 16
NW = NC * NS
L = 16
TPW = T // NW

BR = 256
BRLOG = 8
NB = 24
NR = NB * BR


def _gating_kernel(flat_ref, gate_w_ref, i1_ref, i2_ref, c1_ref, c2_ref,
                   loss_ref, wcounts_ref, bexp_ref, bact_ref, bsrc_ref):
    flat = flat_ref[...]
    logits = jnp.dot(flat, gate_w_ref[...], preferred_element_type=jnp.float32)
    m = jnp.max(logits, axis=-1, keepdims=True)
    ex = jnp.exp(logits - m)
    gates = ex / jnp.sum(ex, axis=-1, keepdims=True)

    eidx = jax.lax.broadcasted_iota(jnp.int32, gates.shape, 1)
    i1 = jnp.argmax(gates, axis=-1, keepdims=True)
    g1 = jnp.max(gates, axis=-1, keepdims=True)
    masked = jnp.where(eidx == i1, -jnp.inf, gates)
    i2 = jnp.argmax(masked, axis=-1, keepdims=True)
    g2 = jnp.max(masked, axis=-1, keepdims=True)

    denom = g1 + g2
    i1_ref[...] = i1.astype(jnp.int32).reshape(T)
    i2_ref[...] = i2.astype(jnp.int32).reshape(T)
    c1_ref[...] = (g1 / denom).reshape(T)
    c2_ref[...] = (g2 / denom).reshape(T)

    sel = ((eidx == i1) | (eidx == i2)).astype(jnp.float32)
    usage = jnp.sum(sel, axis=0) / gates.shape[0]
    loss = jnp.mean((usage - 1.0 / E) ** 2) * E * 0.01
    loss_ref[...] = jnp.reshape(loss, (1, 1))

    eidx16 = jax.lax.broadcasted_iota(jnp.int32, (T, L), 1)
    sel16 = ((eidx16 == i1) | (eidx16 == i2)).astype(jnp.float32)
    wc = jnp.sum(sel16.reshape(NW, TPW, L), axis=1)
    wcounts_ref[...] = wc.astype(jnp.int32)

    total = jnp.sum(wc, axis=0, keepdims=True)
    nblk = jnp.floor((total + (BR - 1)) * (1.0 / BR))
    rr = jax.lax.broadcasted_iota(jnp.int32, (L, L), 0)
    cc = jax.lax.broadcasted_iota(jnp.int32, (L, L), 1)
    tril = (rr < cc).astype(jnp.float32)
    cume = jnp.dot(nblk, tril, preferred_element_type=jnp.float32)
    used = jnp.sum(nblk, axis=-1, keepdims=True)
    ii = jax.lax.broadcasted_iota(jnp.int32, (NB, 1), 0).astype(jnp.float32)
    ge = (ii >= cume).astype(jnp.float32)
    bexp_ref[...] = (jnp.sum(ge[:, :E], axis=1, keepdims=True) - 1.0
                     ).astype(jnp.int32).reshape(NB)
    bact_ref[...] = (ii < used).astype(jnp.int32).reshape(NB)
    bsrc_ref[...] = jnp.minimum(ii, used - 1.0).astype(jnp.int32).reshape(NB)


def _sc_wid():
    return lax.axis_index("s") * NC + lax.axis_index("c")


def _lanes():
    return lax.broadcasted_iota(jnp.int32, (L,), 0)


def _extract(vec, lane):
    return jnp.sum(jnp.where(_lanes() == lane, vec, vec.dtype.type(0)))


def _b1b_kernel(i1_hbm, i2_hbm, x_hbm, wcounts_hbm,
                xg_hbm, pos1_hbm, pos2_hbm,
                i1_v, i2_v, cnt_all_v, pos1_v, pos2_v,
                xrows_v, sem):
    wid = _sc_wid()
    base = wid * TPW
    lanes = _lanes()

    cps = [
        pltpu.async_copy(wcounts_hbm, cnt_all_v, sem),
        pltpu.async_copy(i1_hbm.at[pl.ds(base, TPW)], i1_v, sem),
        pltpu.async_copy(i2_hbm.at[pl.ds(base, TPW)], i2_v, sem),
        pltpu.async_copy(x_hbm.at[pl.ds(base, TPW)], xrows_v, sem),
    ]
    for cp in cps:
        cp.wait()

    total = jnp.zeros((L,), jnp.int32)
    pref = jnp.zeros((L,), jnp.int32)
    for w in range(NW):
        row = cnt_all_v[w]
        total = total + row
        pref = pref + jnp.where(wid > w, row, 0)
    nblk = (total + (BR - 1)) >> BRLOG
    cum = plsc.cumsum(nblk)
    cume = cum - nblk
    off = cume * BR + pref

    def assign(src_ref, pos_ref, off):
        for v in range(TPW // L):
            vec = src_ref[pl.ds(v * L, L)]
            pos = jnp.zeros((L,), jnp.int32)
            for e in range(E):
                m = vec == e
                mi = jnp.where(m, 1, 0)
                pr = plsc.cumsum(mi)
                base_e = _extract(off, e)
                pos = jnp.where(m, base_e + pr - 1, pos)
                off = off + jnp.where(lanes == e, jnp.sum(mi), 0)
            pos_ref[pl.ds(v * L, L)] = pos
        return off

    off = assign(i1_v, pos1_v, off)
    off = assign(i2_v, pos2_v, off)

    cps = [
        pltpu.async_copy(pos1_v, pos1_hbm.at[pl.ds(base, TPW)], sem),
        pltpu.async_copy(pos2_v, pos2_hbm.at[pl.ds(base, TPW)], sem),
        pltpu.async_copy(xrows_v, xg_hbm.at[pos1_v], sem),
        pltpu.async_copy(xrows_v, xg_hbm.at[pos2_v], sem),
    ]
    for cp in cps:
        cp.wait()


def _ffn_kernel(be_ref, act_ref, bsrc_ref, xg_ref, w1_ref, b1_ref, w2_ref,
                b2_ref, out_ref):
    i = pl.program_id(0)

    @pl.when(act_ref[i] == 1)
    def _():
        x = xg_ref[...]
        h = jnp.dot(x, w1_ref[0], preferred_element_type=jnp.float32)
        h = h + b1_ref[0]
        h = 0.5 * h * (1.0 + lax.erf(h * 0.7071067811865476))
        eo = jnp.dot(h, w2_ref[0], preferred_element_type=jnp.float32)
        out_ref[...] = eo + b2_ref[0]


def _comb_kernel(pos1_hbm, pos2_hbm, c1_hbm, c2_hbm, outg_hbm, out_hbm,
                 pos1_v, pos2_v, c1_v, c2_v, rows1_v, rows2_v, sem):
    wid = _sc_wid()
    base = wid * TPW
    lanes = _lanes()
    cps = [
        pltpu.async_copy(pos1_hbm.at[pl.ds(base, TPW)], pos1_v, sem),
        pltpu.async_copy(pos2_hbm.at[pl.ds(base, TPW)], pos2_v, sem),
        pltpu.async_copy(c1_hbm.at[pl.ds(base, TPW)], c1_v, sem),
        pltpu.async_copy(c2_hbm.at[pl.ds(base, TPW)], c2_v, sem),
    ]
    for cp in cps:
        cp.wait()
    HF = TPW // 2
    gathers = []
    for h in range(2):
        hs = pl.ds(h * HF, HF)
        gathers.append((
            pltpu.async_copy(outg_hbm.at[pos1_v.at[hs]], rows1_v.at[hs], sem),
            pltpu.async_copy(outg_hbm.at[pos2_v.at[hs]], rows2_v.at[hs], sem),
        ))

    def body(j, _):
        grp = pl.ds((j // L) * L, L)
        lane = j % L
        s1 = jnp.sum(jnp.where(lanes == lane, c1_v[grp], 0.0))
        s2 = jnp.sum(jnp.where(lanes == lane, c2_v[grp], 0.0))
        for v in range(DIM // L):
            sl = pl.ds(v * L, L)
            rows1_v[j, sl] = rows1_v[j, sl] * s1 + rows2_v[j, sl] * s2
        return 0

    outs = []
    for h in range(2):
        g1, g2 = gathers[h]
        g1.wait()
        g2.wait()
        lax.fori_loop(h * HF, (h + 1) * HF, body, 0)
        outs.append(pltpu.async_copy(
            rows1_v.at[pl.ds(h * HF, HF)],
            out_hbm.at[pl.ds(base + h * HF, HF)], sem))
    for cp in outs:
        cp.wait()


@jax.jit
def kernel(x, gate_w, w1, b1, w2, b2):
    Bt, S, D = x.shape
    flat = x.reshape(T, D)

    i1, i2, c1, c2, loss, wcounts, bexp, bact, bsrc = pl.pallas_call(
        _gating_kernel,
        out_shape=(
            jax.ShapeDtypeStruct((T,), jnp.int32),
            jax.ShapeDtypeStruct((T,), jnp.int32),
            jax.ShapeDtypeStruct((T,), jnp.float32),
            jax.ShapeDtypeStruct((T,), jnp.float32),
            jax.ShapeDtypeStruct((1, 1), jnp.float32),
            jax.ShapeDtypeStruct((NW, L), jnp.int32),
            jax.ShapeDtypeStruct((NB,), jnp.int32),
            jax.ShapeDtypeStruct((NB,), jnp.int32),
            jax.ShapeDtypeStruct((NB,), jnp.int32),
        ),
    )(flat, gate_w)

    mesh = plsc.VectorSubcoreMesh(core_axis_name="c", subcore_axis_name="s")
    sc_params = pltpu.CompilerParams(needs_layout_passes=False)

    xg, pos1, pos2 = pl.kernel(
        _b1b_kernel,
        out_type=(
            jax.ShapeDtypeStruct((NR, DIM), jnp.float32),
            jax.ShapeDtypeStruct((T,), jnp.int32),
            jax.ShapeDtypeStruct((T,), jnp.int32),
        ),
        mesh=mesh,
        compiler_params=sc_params,
        scratch_types=[
            pltpu.VMEM((TPW,), jnp.int32),
            pltpu.VMEM((TPW,), jnp.int32),
            pltpu.VMEM((NW, L), jnp.int32),
            pltpu.VMEM((TPW,), jnp.int32),
            pltpu.VMEM((TPW,), jnp.int32),
            pltpu.VMEM((TPW, DIM), jnp.float32),
            pltpu.SemaphoreType.DMA,
        ],
    )(i1, i2, flat, wcounts)

    grid_spec = pltpu.PrefetchScalarGridSpec(
        num_scalar_prefetch=3,
        grid=(NB,),
        in_specs=[
            pl.BlockSpec((BR, DIM), lambda i, be, act, bs: (bs[i], 0)),
            pl.BlockSpec((1, DIM, H), lambda i, be, act, bs: (be[i], 0, 0)),
            pl.BlockSpec((1, 1, H), lambda i, be, act, bs: (be[i], 0, 0)),
            pl.BlockSpec((1, H, DIM), lambda i, be, act, bs: (be[i], 0, 0)),
            pl.BlockSpec((1, 1, DIM), lambda i, be, act, bs: (be[i], 0, 0)),
        ],
        out_specs=pl.BlockSpec((BR, DIM), lambda i, be, act, bs: (bs[i], 0)),
    )
    outg = pl.pallas_call(
        _ffn_kernel,
        grid_spec=grid_spec,
        out_shape=jax.ShapeDtypeStruct((NR, DIM), jnp.float32),
    )(bexp, bact, bsrc, xg, w1, b1.reshape(E, 1, H), w2, b2.reshape(E, 1, DIM))

    out2d = pl.kernel(
        _comb_kernel,
        out_type=jax.ShapeDtypeStruct((T, DIM), jnp.float32),
        mesh=mesh,
        compiler_params=sc_params,
        scratch_types=[
            pltpu.VMEM((TPW,), jnp.int32),
            pltpu.VMEM((TPW,), jnp.int32),
            pltpu.VMEM((TPW,), jnp.float32),
            pltpu.VMEM((TPW,), jnp.float32),
            pltpu.VMEM((TPW, DIM), jnp.float32),
            pltpu.VMEM((TPW, DIM), jnp.float32),
            pltpu.SemaphoreType.DMA,
        ],
    )(pos1, pos2, c1, c2, outg)

    return out2d.reshape(Bt, S, D), loss[0, 0]
